# baseline (speedup 1.0000x reference)
import math
import functools

import jax
import jax.numpy as jnp
from jax import lax
from jax.experimental import pallas as pl
from jax.experimental.pallas import tpu as pltpu
from jax.experimental.pallas import tpu_sc as plsc

NN = 10000
EE = 320000
FF = 128
LL = 8
ALPHA_C = 0.1
THETA_C = 0.5
EPS_C = 1e-5

NC = 2
NS = 16
NW = NC * NS
BLK = 128
NBLK = 80
EP = NW * NBLK * BLK
NPAD_ROWS = 80
NROWS = NN + NPAD_ROWS
RPT = 632
RPT_LAST = NROWS - (NS - 1) * RPT

BK = 1024
GRID = (NN + BK - 1) // BK

_mesh = plsc.VectorSubcoreMesh(core_axis_name="c", subcore_axis_name="s")


def _zero_rows_f32(buf):
    z = jnp.zeros((16,), jnp.float32)

    def body(i, _):
        for k in range(8):
            buf[i, pl.ds(k * 16, 16)] = z
        return 0

    lax.fori_loop(0, buf.shape[0], body, 0)


def _fill_1d(buf, val):
    v = jnp.full((16,), val, jnp.float32)
    for k in range(buf.shape[0] // 16):
        buf[pl.ds(k * 16, 16)] = v


def _per_tile_rows(s, fn):
    @pl.when(s < NS - 1)
    def _():
        fn(RPT)

    @pl.when(s == NS - 1)
    def _():
        fn(RPT_LAST)


@functools.partial(
    pl.kernel,
    out_type=jax.ShapeDtypeStruct((NC * NROWS,), jnp.float32),
    mesh=_mesh,
    scratch_types=[
        pltpu.VMEM((NBLK, BLK), jnp.int32),
        pltpu.VMEM((BLK,), jnp.float32),
        pltpu.VMEM((BLK,), jnp.float32),
        pltpu.VMEM((RPT,), jnp.float32),
        pltpu.VMEM_SHARED((NROWS,), jnp.float32),
        pltpu.SemaphoreType.DMA,
    ],
)
def _deg_kernel(colp_hbm, out_hbm, coli_v, ones_v, zeros_v, wb_v, acc_sh, sem):
    c = lax.axis_index("c")
    s = lax.axis_index("s")
    w = c * NS + s
    base = s * RPT

    pltpu.sync_copy(colp_hbm.at[w], coli_v)
    _fill_1d(ones_v, 1.0)
    _fill_1d(zeros_v, 0.0)

    def zero_rows(nrows):
        nfull, rem = divmod(nrows, BLK)
        for k in range(nfull):
            pltpu.sync_copy(zeros_v, acc_sh.at[pl.ds(base + k * BLK, BLK)])
        if rem:
            pltpu.sync_copy(zeros_v.at[pl.ds(0, rem)],
                            acc_sh.at[pl.ds(base + nfull * BLK, rem)])

    _per_tile_rows(s, zero_rows)
    plsc.subcore_barrier()

    def body(j, _):
        pltpu.async_copy(ones_v, acc_sh.at[coli_v.at[j]], sem, add=True)
        return 0

    lax.fori_loop(0, NBLK, body, 0)

    def drain(j, _):
        pltpu.make_async_copy(ones_v, acc_sh.at[coli_v.at[0]], sem).wait()
        return 0

    lax.fori_loop(0, NBLK, drain, 0)
    plsc.subcore_barrier()

    def writeback(nrows):
        pltpu.sync_copy(acc_sh.at[pl.ds(base, nrows)], wb_v.at[pl.ds(0, nrows)])
        pltpu.sync_copy(wb_v.at[pl.ds(0, nrows)],
                        out_hbm.at[pl.ds(c * NROWS + base, nrows)])

    _per_tile_rows(s, writeback)


@functools.partial(
    pl.kernel,
    out_type=jax.ShapeDtypeStruct((NC, NROWS, FF), jnp.float32),
    mesh=_mesh,
    scratch_types=[
        pltpu.VMEM((8, BLK), jnp.int32),
        pltpu.VMEM((BLK, FF), jnp.float32),
        pltpu.VMEM((BLK, FF), jnp.float32),
        pltpu.VMEM((BLK, FF), jnp.float32),
        pltpu.VMEM_SHARED((NROWS, FF), jnp.float32),
        pltpu.SemaphoreType.DMA,
        pltpu.SemaphoreType.DMA,
        pltpu.SemaphoreType.DMA,
        pltpu.SemaphoreType.DMA,
        pltpu.SemaphoreType.DMA,
        pltpu.SemaphoreType.DMA,
        pltpu.SemaphoreType.DMA,
        pltpu.SemaphoreType.DMA,
        pltpu.SemaphoreType.DMA,
        pltpu.SemaphoreType.DMA,
    ],
)
def _spmm_kernel(hs_hbm, rc_hbm, out_hbm,
                 idxb, gbuf0, gbuf1, gbuf2, acc_sh,
                 i0, i1, i2, i3, g0, g1, g2, s0, s1, s2):
    c = lax.axis_index("c")
    s = lax.axis_index("s")
    w = c * NS + s
    base = s * RPT
    gbase = w * NBLK

    isems = (i0, i1, i2, i3)
    gsems = (g0, g1, g2)
    ssems = (s0, s1, s2)
    bufs = (gbuf0, gbuf1, gbuf2)

    def start_idx(j, k):
        pltpu.async_copy(rc_hbm.at[gbase + j], idxb.at[pl.ds(2 * k, 2)],
                         isems[k])

    def wait_idx(k):
        pltpu.make_async_copy(rc_hbm.at[0], idxb.at[pl.ds(2 * k, 2)],
                              isems[k]).wait()

    def start_gather(k, b):
        pltpu.async_copy(hs_hbm.at[idxb.at[2 * k]], bufs[b], gsems[b])

    def wait_gather(k, b):
        pltpu.make_async_copy(hs_hbm.at[idxb.at[2 * k]], bufs[b],
                              gsems[b]).wait()

    def start_scatter(k, b):
        pltpu.async_copy(bufs[b], acc_sh.at[idxb.at[2 * k + 1]],
                         ssems[b], add=True)

    def wait_scatter(k, b):
        pltpu.make_async_copy(bufs[b], acc_sh.at[idxb.at[2 * k + 1]],
                              ssems[b]).wait()

    for j in range(3):
        start_idx(j, j)

    _zero_rows_f32(gbuf1)

    def zero_rows(nrows):
        nfull, rem = divmod(nrows, BLK)
        for k in range(nfull):
            pltpu.async_copy(gbuf1, acc_sh.at[pl.ds(base + k * BLK, BLK)], g0)
        if rem:
            pltpu.async_copy(gbuf1.at[pl.ds(0, rem)],
                             acc_sh.at[pl.ds(base + nfull * BLK, rem)], g0)
        for k in range(nfull):
            pltpu.make_async_copy(
                gbuf1, acc_sh.at[pl.ds(base + k * BLK, BLK)], g0).wait()
        if rem:
            pltpu.make_async_copy(
                gbuf1.at[pl.ds(0, rem)],
                acc_sh.at[pl.ds(base + nfull * BLK, rem)], g0).wait()

    _per_tile_rows(s, zero_rows)
    plsc.subcore_barrier()

    wait_idx(0)
    start_gather(0, 0)
    wait_idx(1)
    start_gather(1, 1)
    wait_gather(0, 0)
    start_scatter(0, 0)
    start_idx(3, 3)
    wait_idx(2)
    start_gather(2, 2)
    wait_gather(1, 1)
    start_scatter(1, 1)
    wait_scatter(0, 0)
    start_idx(4, 0)
    wait_idx(3)
    start_gather(3, 0)

    def step(j, k, b):
        wait_gather(k, b)
        start_scatter(k, b)
        wait_scatter((k + 3) % 4, (b + 2) % 3)
        if isinstance(j, int) and j + 3 >= NBLK:
            pass
        else:
            start_idx(j + 3, (k + 3) % 4)
        wait_idx((k + 2) % 4)
        start_gather((k + 2) % 4, (b + 2) % 3)

    def body12(j0, _):
        for u in range(12):
            step(j0 + u, (2 + u) % 4, (2 + u) % 3)
        return 0

    lax.fori_loop(0, (NBLK - 8) // 12, lambda t, x: body12(2 + 12 * t, x), 0)
    for j in range(NBLK - 6, NBLK - 2):
        step(j, j % 4, j % 3)

    for j in (NBLK - 2, NBLK - 1):
        wait_gather(j % 4, j % 3)
        start_scatter(j % 4, j % 3)
        wait_scatter((j - 1) % 4, (j - 1) % 3)
    wait_scatter((NBLK - 1) % 4, (NBLK - 1) % 3)

    plsc.subcore_barrier()

    def writeback(nrows):
        pltpu.sync_copy(acc_sh.at[pl.ds(base, nrows)],
                        out_hbm.at[c, pl.ds(base, nrows)])

    _per_tile_rows(s, writeback)


def _first_body(x_ref, w0t_ref, b0_ref, deg_ref, h0_ref, hs_ref):
    deg = deg_ref[0] + deg_ref[1]
    dis = jnp.where(deg > 0.0, lax.rsqrt(deg), 0.0)
    h = jnp.dot(x_ref[...], w0t_ref[...],
                preferred_element_type=jnp.float32,
                precision=lax.Precision.HIGHEST) + b0_ref[...]
    h = jnp.maximum(h, 0.0)
    h0_ref[...] = h.astype(jnp.bfloat16)
    hs_ref[...] = dis * h


def _layer_body(beta_l, last, sp_ref, h0_ref, deg_ref, w_ref, g_ref, bt_ref,
                w1t_ref, b1_ref, out_ref):
    deg = deg_ref[0] + deg_ref[1]
    dis = jnp.where(deg > 0.0, lax.rsqrt(deg), 0.0)
    ssum = sp_ref[0] + sp_ref[1]
    h0v = h0_ref[...].astype(jnp.float32)
    t = (1.0 - ALPHA_C) * (dis * ssum) + ALPHA_C * h0v
    tw = jnp.dot(t, w_ref[...], preferred_element_type=jnp.float32,
                 precision=lax.Precision.HIGHEST)
    u = (1.0 - beta_l) * t + beta_l * tw
    u = g_ref[...] * u * (1.0 / math.sqrt(1.0 + EPS_C)) + bt_ref[...]
    h = jnp.maximum(u, 0.0)
    if last:
        out_ref[...] = jnp.dot(h, w1t_ref[...],
                               preferred_element_type=jnp.float32,
                               precision=lax.Precision.HIGHEST) + b1_ref[...]
    else:
        out_ref[...] = dis * h


def _row_spec(feat):
    return pl.BlockSpec((BK, feat), lambda i: (i, 0))


_FULL = pl.BlockSpec((FF, FF), lambda i: (0, 0))
_ROWV = pl.BlockSpec((1, FF), lambda i: (0, 0))
_DEGS = pl.BlockSpec((NC, BK, 1), lambda i: (0, i, 0))
_SPS = pl.BlockSpec((NC, BK, FF), lambda i: (0, i, 0))


def kernel(x, edge_index, W0, b0, W_convs, gammas, betas, W1, b1):
    row = edge_index[0].astype(jnp.int32)
    col = edge_index[1].astype(jnp.int32)
    pad = EP - EE
    pad_row = (jnp.arange(pad, dtype=jnp.int32) * 7) % NN
    pad_col = NN + (jnp.arange(pad, dtype=jnp.int32) % NPAD_ROWS)
    rowp = jnp.concatenate([row, pad_row])
    colp = jnp.concatenate([col, pad_col])
    rc3 = jnp.stack([rowp.reshape(NW * NBLK, BLK),
                     colp.reshape(NW * NBLK, BLK)], axis=1)
    colp3 = colp.reshape(NW, NBLK, BLK)

    degp = _deg_kernel(colp3)
    degp3 = degp.reshape(NC, NROWS, 1)

    h0, hs = pl.pallas_call(
        _first_body,
        grid=(GRID,),
        in_specs=[_row_spec(FF), _FULL, _ROWV, _DEGS],
        out_specs=[_row_spec(FF), _row_spec(FF)],
        out_shape=[jax.ShapeDtypeStruct((NN, FF), jnp.bfloat16),
                   jax.ShapeDtypeStruct((NN, FF), jnp.float32)],
    )(x, W0.T, b0.reshape(1, FF), degp3)

    dummy = jnp.zeros((FF, FF), jnp.float32)
    dummyv = jnp.zeros((1, FF), jnp.float32)
    w1t = W1.T
    b1v = b1.reshape(1, FF)

    for l in range(LL):
        sp = _spmm_kernel(hs, rc3)
        beta_l = math.log(THETA_C / (l + 1) + 1.0)
        last = l == LL - 1
        hs = pl.pallas_call(
            functools.partial(_layer_body, beta_l, last),
            grid=(GRID,),
            in_specs=[_SPS, _row_spec(FF), _DEGS, _FULL, _ROWV, _ROWV,
                      _FULL, _ROWV],
            out_specs=_row_spec(FF),
            out_shape=jax.ShapeDtypeStruct((NN, FF), jnp.float32),
        )(sp, h0, degp3, W_convs[l],
          gammas[l].reshape(1, FF), betas[l].reshape(1, FF),
          w1t if last else dummy, b1v if last else dummyv)
    return hs


# speedup vs baseline: 1.0212x; 1.0212x over previous
import math
import functools

import jax
import jax.numpy as jnp
from jax import lax
from jax.experimental import pallas as pl
from jax.experimental.pallas import tpu as pltpu
from jax.experimental.pallas import tpu_sc as plsc

NN = 10000
EE = 320000
FF = 128
LL = 8
ALPHA_C = 0.1
THETA_C = 0.5
EPS_C = 1e-5

NC = 2
NS = 16
NW = NC * NS
BLK = 128
NBLK = 80
EP = NW * NBLK * BLK
NPAD_ROWS = 80
NROWS = NN + NPAD_ROWS
RPT = 632
RPT_LAST = NROWS - (NS - 1) * RPT

BK = 2048
GRID = (NN + BK - 1) // BK

_mesh = plsc.VectorSubcoreMesh(core_axis_name="c", subcore_axis_name="s")


def _zero_rows_f32(buf):
    z = jnp.zeros((16,), jnp.float32)

    def body(i, _):
        for k in range(8):
            buf[i, pl.ds(k * 16, 16)] = z
        return 0

    lax.fori_loop(0, buf.shape[0], body, 0)


def _fill_1d(buf, val):
    v = jnp.full((16,), val, jnp.float32)
    for k in range(buf.shape[0] // 16):
        buf[pl.ds(k * 16, 16)] = v


def _per_tile_rows(s, fn):
    @pl.when(s < NS - 1)
    def _():
        fn(RPT)

    @pl.when(s == NS - 1)
    def _():
        fn(RPT_LAST)


@functools.partial(
    pl.kernel,
    out_type=jax.ShapeDtypeStruct((NC * NROWS,), jnp.float32),
    mesh=_mesh,
    scratch_types=[
        pltpu.VMEM((NBLK, BLK), jnp.int32),
        pltpu.VMEM((BLK,), jnp.float32),
        pltpu.VMEM((BLK,), jnp.float32),
        pltpu.VMEM((RPT,), jnp.float32),
        pltpu.VMEM_SHARED((NROWS,), jnp.float32),
        pltpu.SemaphoreType.DMA,
    ],
)
def _deg_kernel(colp_hbm, out_hbm, coli_v, ones_v, zeros_v, wb_v, acc_sh, sem):
    c = lax.axis_index("c")
    s = lax.axis_index("s")
    w = c * NS + s
    base = s * RPT

    pltpu.sync_copy(colp_hbm.at[w], coli_v)
    _fill_1d(ones_v, 1.0)
    _fill_1d(zeros_v, 0.0)

    def zero_rows(nrows):
        nfull, rem = divmod(nrows, BLK)
        for k in range(nfull):
            pltpu.sync_copy(zeros_v, acc_sh.at[pl.ds(base + k * BLK, BLK)])
        if rem:
            pltpu.sync_copy(zeros_v.at[pl.ds(0, rem)],
                            acc_sh.at[pl.ds(base + nfull * BLK, rem)])

    _per_tile_rows(s, zero_rows)
    plsc.subcore_barrier()

    def body(j, _):
        pltpu.async_copy(ones_v, acc_sh.at[coli_v.at[j]], sem, add=True)
        return 0

    lax.fori_loop(0, NBLK, body, 0)

    def drain(j, _):
        pltpu.make_async_copy(ones_v, acc_sh.at[coli_v.at[0]], sem).wait()
        return 0

    lax.fori_loop(0, NBLK, drain, 0)
    plsc.subcore_barrier()

    def writeback(nrows):
        pltpu.sync_copy(acc_sh.at[pl.ds(base, nrows)], wb_v.at[pl.ds(0, nrows)])
        pltpu.sync_copy(wb_v.at[pl.ds(0, nrows)],
                        out_hbm.at[pl.ds(c * NROWS + base, nrows)])

    _per_tile_rows(s, writeback)


@functools.partial(
    pl.kernel,
    out_type=jax.ShapeDtypeStruct((NC, NROWS, FF), jnp.float32),
    mesh=_mesh,
    scratch_types=[
        pltpu.VMEM((8, BLK), jnp.int32),
        pltpu.VMEM((BLK, FF), jnp.float32),
        pltpu.VMEM((BLK, FF), jnp.float32),
        pltpu.VMEM((BLK, FF), jnp.float32),
        pltpu.VMEM_SHARED((NROWS, FF), jnp.float32),
        pltpu.SemaphoreType.DMA,
        pltpu.SemaphoreType.DMA,
        pltpu.SemaphoreType.DMA,
        pltpu.SemaphoreType.DMA,
        pltpu.SemaphoreType.DMA,
        pltpu.SemaphoreType.DMA,
        pltpu.SemaphoreType.DMA,
        pltpu.SemaphoreType.DMA,
        pltpu.SemaphoreType.DMA,
        pltpu.SemaphoreType.DMA,
    ],
)
def _spmm_kernel(hs_hbm, rc_hbm, out_hbm,
                 idxb, gbuf0, gbuf1, gbuf2, acc_sh,
                 i0, i1, i2, i3, g0, g1, g2, s0, s1, s2):
    c = lax.axis_index("c")
    s = lax.axis_index("s")
    w = c * NS + s
    base = s * RPT
    gbase = w * NBLK

    isems = (i0, i1, i2, i3)
    gsems = (g0, g1, g2)
    ssems = (s0, s1, s2)
    bufs = (gbuf0, gbuf1, gbuf2)

    def start_idx(j, k):
        pltpu.async_copy(rc_hbm.at[gbase + j], idxb.at[pl.ds(2 * k, 2)],
                         isems[k])

    def wait_idx(k):
        pltpu.make_async_copy(rc_hbm.at[0], idxb.at[pl.ds(2 * k, 2)],
                              isems[k]).wait()

    def start_gather(k, b):
        pltpu.async_copy(hs_hbm.at[idxb.at[2 * k]], bufs[b], gsems[b])

    def wait_gather(k, b):
        pltpu.make_async_copy(hs_hbm.at[idxb.at[2 * k]], bufs[b],
                              gsems[b]).wait()

    def start_scatter(k, b):
        pltpu.async_copy(bufs[b], acc_sh.at[idxb.at[2 * k + 1]],
                         ssems[b], add=True)

    def wait_scatter(k, b):
        pltpu.make_async_copy(bufs[b], acc_sh.at[idxb.at[2 * k + 1]],
                              ssems[b]).wait()

    for j in range(3):
        start_idx(j, j)

    _zero_rows_f32(gbuf1)

    def zero_rows(nrows):
        nfull, rem = divmod(nrows, BLK)
        for k in range(nfull):
            pltpu.async_copy(gbuf1, acc_sh.at[pl.ds(base + k * BLK, BLK)], g0)
        if rem:
            pltpu.async_copy(gbuf1.at[pl.ds(0, rem)],
                             acc_sh.at[pl.ds(base + nfull * BLK, rem)], g0)
        for k in range(nfull):
            pltpu.make_async_copy(
                gbuf1, acc_sh.at[pl.ds(base + k * BLK, BLK)], g0).wait()
        if rem:
            pltpu.make_async_copy(
                gbuf1.at[pl.ds(0, rem)],
                acc_sh.at[pl.ds(base + nfull * BLK, rem)], g0).wait()

    _per_tile_rows(s, zero_rows)
    plsc.subcore_barrier()

    wait_idx(0)
    start_gather(0, 0)
    wait_idx(1)
    start_gather(1, 1)
    wait_gather(0, 0)
    start_scatter(0, 0)
    start_idx(3, 3)
    wait_idx(2)
    start_gather(2, 2)
    wait_gather(1, 1)
    start_scatter(1, 1)
    wait_scatter(0, 0)
    start_idx(4, 0)
    wait_idx(3)
    start_gather(3, 0)

    def step(j, k, b):
        wait_gather(k, b)
        start_scatter(k, b)
        wait_scatter((k + 3) % 4, (b + 2) % 3)
        if isinstance(j, int) and j + 3 >= NBLK:
            pass
        else:
            start_idx(j + 3, (k + 3) % 4)
        wait_idx((k + 2) % 4)
        start_gather((k + 2) % 4, (b + 2) % 3)

    def body12(j0, _):
        for u in range(12):
            step(j0 + u, (2 + u) % 4, (2 + u) % 3)
        return 0

    lax.fori_loop(0, (NBLK - 8) // 12, lambda t, x: body12(2 + 12 * t, x), 0)
    for j in range(NBLK - 6, NBLK - 2):
        step(j, j % 4, j % 3)

    for j in (NBLK - 2, NBLK - 1):
        wait_gather(j % 4, j % 3)
        start_scatter(j % 4, j % 3)
        wait_scatter((j - 1) % 4, (j - 1) % 3)
    wait_scatter((NBLK - 1) % 4, (NBLK - 1) % 3)

    plsc.subcore_barrier()

    def writeback(nrows):
        pltpu.sync_copy(acc_sh.at[pl.ds(base, nrows)],
                        out_hbm.at[c, pl.ds(base, nrows)])

    _per_tile_rows(s, writeback)


def _first_body(x_ref, w0t_ref, b0_ref, deg_ref, h0_ref, hs_ref):
    deg = deg_ref[0] + deg_ref[1]
    dis = jnp.where(deg > 0.0, lax.rsqrt(deg), 0.0)
    h = jnp.dot(x_ref[...], w0t_ref[...],
                preferred_element_type=jnp.float32,
                precision=lax.Precision.HIGHEST) + b0_ref[...]
    h = jnp.maximum(h, 0.0)
    h0_ref[...] = h.astype(jnp.bfloat16)
    hs_ref[...] = dis * h


def _layer_body(beta_l, last, sp_ref, h0_ref, deg_ref, w_ref, g_ref, bt_ref,
                w1t_ref, b1_ref, out_ref):
    deg = deg_ref[0] + deg_ref[1]
    dis = jnp.where(deg > 0.0, lax.rsqrt(deg), 0.0)
    ssum = sp_ref[0] + sp_ref[1]
    h0v = h0_ref[...].astype(jnp.float32)
    t = (1.0 - ALPHA_C) * (dis * ssum) + ALPHA_C * h0v
    tw = jnp.dot(t, w_ref[...], preferred_element_type=jnp.float32,
                 precision=lax.Precision.HIGHEST)
    u = (1.0 - beta_l) * t + beta_l * tw
    u = g_ref[...] * u * (1.0 / math.sqrt(1.0 + EPS_C)) + bt_ref[...]
    h = jnp.maximum(u, 0.0)
    if last:
        out_ref[...] = jnp.dot(h, w1t_ref[...],
                               preferred_element_type=jnp.float32,
                               precision=lax.Precision.HIGHEST) + b1_ref[...]
    else:
        out_ref[...] = dis * h


def _row_spec(feat):
    return pl.BlockSpec((BK, feat), lambda i: (i, 0))


_FULL = pl.BlockSpec((FF, FF), lambda i: (0, 0))
_ROWV = pl.BlockSpec((1, FF), lambda i: (0, 0))
_DEGS = pl.BlockSpec((NC, BK, 1), lambda i: (0, i, 0))
_SPS = pl.BlockSpec((NC, BK, FF), lambda i: (0, i, 0))


def kernel(x, edge_index, W0, b0, W_convs, gammas, betas, W1, b1):
    row = edge_index[0].astype(jnp.int32)
    col = edge_index[1].astype(jnp.int32)
    pad = EP - EE
    pad_row = (jnp.arange(pad, dtype=jnp.int32) * 7) % NN
    pad_col = NN + (jnp.arange(pad, dtype=jnp.int32) % NPAD_ROWS)
    rowp = jnp.concatenate([row, pad_row])
    colp = jnp.concatenate([col, pad_col])
    rc3 = jnp.stack([rowp.reshape(NW * NBLK, BLK),
                     colp.reshape(NW * NBLK, BLK)], axis=1)
    colp3 = colp.reshape(NW, NBLK, BLK)

    degp = _deg_kernel(colp3)
    degp3 = degp.reshape(NC, NROWS, 1)

    h0, hs = pl.pallas_call(
        _first_body,
        grid=(GRID,),
        in_specs=[_row_spec(FF), _FULL, _ROWV, _DEGS],
        out_specs=[_row_spec(FF), _row_spec(FF)],
        out_shape=[jax.ShapeDtypeStruct((NN, FF), jnp.bfloat16),
                   jax.ShapeDtypeStruct((NN, FF), jnp.float32)],
    )(x, W0.T, b0.reshape(1, FF), degp3)

    dummy = jnp.zeros((FF, FF), jnp.float32)
    dummyv = jnp.zeros((1, FF), jnp.float32)
    w1t = W1.T
    b1v = b1.reshape(1, FF)

    for l in range(LL):
        sp = _spmm_kernel(hs, rc3)
        beta_l = math.log(THETA_C / (l + 1) + 1.0)
        last = l == LL - 1
        hs = pl.pallas_call(
            functools.partial(_layer_body, beta_l, last),
            grid=(GRID,),
            in_specs=[_SPS, _row_spec(FF), _DEGS, _FULL, _ROWV, _ROWV,
                      _FULL, _ROWV],
            out_specs=_row_spec(FF),
            out_shape=jax.ShapeDtypeStruct((NN, FF), jnp.float32),
        )(sp, h0, degp3, W_convs[l],
          gammas[l].reshape(1, FF), betas[l].reshape(1, FF),
          w1t if last else dummy, b1v if last else dummyv)
    return hs


# speedup vs baseline: 1.0220x; 1.0007x over previous
import math
import functools

import jax
import jax.numpy as jnp
from jax import lax
from jax.experimental import pallas as pl
from jax.experimental.pallas import tpu as pltpu
from jax.experimental.pallas import tpu_sc as plsc

NN = 10000
EE = 320000
FF = 128
LL = 8
ALPHA_C = 0.1
THETA_C = 0.5
EPS_C = 1e-5

NC = 2
NS = 16
NW = NC * NS
BLK = 128
NBLK = 80
EP = NW * NBLK * BLK
NPAD_ROWS = 80
NROWS = NN + NPAD_ROWS
RPT = 632
RPT_LAST = NROWS - (NS - 1) * RPT

BK = 2560
GRID = (NN + BK - 1) // BK

_mesh = plsc.VectorSubcoreMesh(core_axis_name="c", subcore_axis_name="s")


def _zero_rows_f32(buf):
    z = jnp.zeros((16,), jnp.float32)

    def body(i, _):
        for k in range(8):
            buf[i, pl.ds(k * 16, 16)] = z
        return 0

    lax.fori_loop(0, buf.shape[0], body, 0)


def _fill_1d(buf, val):
    v = jnp.full((16,), val, jnp.float32)
    for k in range(buf.shape[0] // 16):
        buf[pl.ds(k * 16, 16)] = v


def _per_tile_rows(s, fn):
    @pl.when(s < NS - 1)
    def _():
        fn(RPT)

    @pl.when(s == NS - 1)
    def _():
        fn(RPT_LAST)


@functools.partial(
    pl.kernel,
    out_type=jax.ShapeDtypeStruct((NC * NROWS,), jnp.float32),
    mesh=_mesh,
    scratch_types=[
        pltpu.VMEM((NBLK, BLK), jnp.int32),
        pltpu.VMEM((BLK,), jnp.float32),
        pltpu.VMEM((BLK,), jnp.float32),
        pltpu.VMEM((RPT,), jnp.float32),
        pltpu.VMEM_SHARED((NROWS,), jnp.float32),
        pltpu.SemaphoreType.DMA,
    ],
)
def _deg_kernel(colp_hbm, out_hbm, coli_v, ones_v, zeros_v, wb_v, acc_sh, sem):
    c = lax.axis_index("c")
    s = lax.axis_index("s")
    w = c * NS + s
    base = s * RPT

    pltpu.sync_copy(colp_hbm.at[w], coli_v)
    _fill_1d(ones_v, 1.0)
    _fill_1d(zeros_v, 0.0)

    def zero_rows(nrows):
        nfull, rem = divmod(nrows, BLK)
        for k in range(nfull):
            pltpu.sync_copy(zeros_v, acc_sh.at[pl.ds(base + k * BLK, BLK)])
        if rem:
            pltpu.sync_copy(zeros_v.at[pl.ds(0, rem)],
                            acc_sh.at[pl.ds(base + nfull * BLK, rem)])

    _per_tile_rows(s, zero_rows)
    plsc.subcore_barrier()

    def body(j, _):
        pltpu.async_copy(ones_v, acc_sh.at[coli_v.at[j]], sem, add=True)
        return 0

    lax.fori_loop(0, NBLK, body, 0)

    def drain(j, _):
        pltpu.make_async_copy(ones_v, acc_sh.at[coli_v.at[0]], sem).wait()
        return 0

    lax.fori_loop(0, NBLK, drain, 0)
    plsc.subcore_barrier()

    def writeback(nrows):
        pltpu.sync_copy(acc_sh.at[pl.ds(base, nrows)], wb_v.at[pl.ds(0, nrows)])
        pltpu.sync_copy(wb_v.at[pl.ds(0, nrows)],
                        out_hbm.at[pl.ds(c * NROWS + base, nrows)])

    _per_tile_rows(s, writeback)


@functools.partial(
    pl.kernel,
    out_type=jax.ShapeDtypeStruct((NC, NROWS, FF), jnp.float32),
    mesh=_mesh,
    scratch_types=[
        pltpu.VMEM((8, BLK), jnp.int32),
        pltpu.VMEM((BLK, FF), jnp.float32),
        pltpu.VMEM((BLK, FF), jnp.float32),
        pltpu.VMEM((BLK, FF), jnp.float32),
        pltpu.VMEM_SHARED((NROWS, FF), jnp.float32),
        pltpu.SemaphoreType.DMA,
        pltpu.SemaphoreType.DMA,
        pltpu.SemaphoreType.DMA,
        pltpu.SemaphoreType.DMA,
        pltpu.SemaphoreType.DMA,
        pltpu.SemaphoreType.DMA,
        pltpu.SemaphoreType.DMA,
        pltpu.SemaphoreType.DMA,
        pltpu.SemaphoreType.DMA,
        pltpu.SemaphoreType.DMA,
    ],
)
def _spmm_kernel(hs_hbm, rc_hbm, out_hbm,
                 idxb, gbuf0, gbuf1, gbuf2, acc_sh,
                 i0, i1, i2, i3, g0, g1, g2, s0, s1, s2):
    c = lax.axis_index("c")
    s = lax.axis_index("s")
    w = c * NS + s
    base = s * RPT
    gbase = w * NBLK

    isems = (i0, i1, i2, i3)
    gsems = (g0, g1, g2)
    ssems = (s0, s1, s2)
    bufs = (gbuf0, gbuf1, gbuf2)

    def start_idx(j, k):
        pltpu.async_copy(rc_hbm.at[gbase + j], idxb.at[pl.ds(2 * k, 2)],
                         isems[k])

    def wait_idx(k):
        pltpu.make_async_copy(rc_hbm.at[0], idxb.at[pl.ds(2 * k, 2)],
                              isems[k]).wait()

    def start_gather(k, b):
        pltpu.async_copy(hs_hbm.at[idxb.at[2 * k]], bufs[b], gsems[b])

    def wait_gather(k, b):
        pltpu.make_async_copy(hs_hbm.at[idxb.at[2 * k]], bufs[b],
                              gsems[b]).wait()

    def start_scatter(k, b):
        pltpu.async_copy(bufs[b], acc_sh.at[idxb.at[2 * k + 1]],
                         ssems[b], add=True)

    def wait_scatter(k, b):
        pltpu.make_async_copy(bufs[b], acc_sh.at[idxb.at[2 * k + 1]],
                              ssems[b]).wait()

    for j in range(3):
        start_idx(j, j)

    _zero_rows_f32(gbuf1)

    def zero_rows(nrows):
        nfull, rem = divmod(nrows, BLK)
        for k in range(nfull):
            pltpu.async_copy(gbuf1, acc_sh.at[pl.ds(base + k * BLK, BLK)], g0)
        if rem:
            pltpu.async_copy(gbuf1.at[pl.ds(0, rem)],
                             acc_sh.at[pl.ds(base + nfull * BLK, rem)], g0)
        for k in range(nfull):
            pltpu.make_async_copy(
                gbuf1, acc_sh.at[pl.ds(base + k * BLK, BLK)], g0).wait()
        if rem:
            pltpu.make_async_copy(
                gbuf1.at[pl.ds(0, rem)],
                acc_sh.at[pl.ds(base + nfull * BLK, rem)], g0).wait()

    _per_tile_rows(s, zero_rows)
    plsc.subcore_barrier()

    wait_idx(0)
    start_gather(0, 0)
    wait_idx(1)
    start_gather(1, 1)
    wait_gather(0, 0)
    start_scatter(0, 0)
    start_idx(3, 3)
    wait_idx(2)
    start_gather(2, 2)
    wait_gather(1, 1)
    start_scatter(1, 1)
    wait_scatter(0, 0)
    start_idx(4, 0)
    wait_idx(3)
    start_gather(3, 0)

    def step(j, k, b):
        wait_gather(k, b)
        start_scatter(k, b)
        wait_scatter((k + 3) % 4, (b + 2) % 3)
        if isinstance(j, int) and j + 3 >= NBLK:
            pass
        else:
            start_idx(j + 3, (k + 3) % 4)
        wait_idx((k + 2) % 4)
        start_gather((k + 2) % 4, (b + 2) % 3)

    def body12(j0, _):
        for u in range(12):
            step(j0 + u, (2 + u) % 4, (2 + u) % 3)
        return 0

    lax.fori_loop(0, (NBLK - 8) // 12, lambda t, x: body12(2 + 12 * t, x), 0)
    for j in range(NBLK - 6, NBLK - 2):
        step(j, j % 4, j % 3)

    for j in (NBLK - 2, NBLK - 1):
        wait_gather(j % 4, j % 3)
        start_scatter(j % 4, j % 3)
        wait_scatter((j - 1) % 4, (j - 1) % 3)
    wait_scatter((NBLK - 1) % 4, (NBLK - 1) % 3)

    plsc.subcore_barrier()

    def writeback(nrows):
        pltpu.sync_copy(acc_sh.at[pl.ds(base, nrows)],
                        out_hbm.at[c, pl.ds(base, nrows)])

    _per_tile_rows(s, writeback)


def _first_body(x_ref, w0t_ref, b0_ref, deg_ref, h0_ref, hs_ref):
    deg = deg_ref[0] + deg_ref[1]
    dis = jnp.where(deg > 0.0, lax.rsqrt(deg), 0.0)
    h = jnp.dot(x_ref[...], w0t_ref[...],
                preferred_element_type=jnp.float32,
                precision=lax.Precision.HIGHEST) + b0_ref[...]
    h = jnp.maximum(h, 0.0)
    h0_ref[...] = h.astype(jnp.bfloat16)
    hs_ref[...] = dis * h


def _layer_body(beta_l, last, sp_ref, h0_ref, deg_ref, w_ref, g_ref, bt_ref,
                w1t_ref, b1_ref, out_ref):
    deg = deg_ref[0] + deg_ref[1]
    dis = jnp.where(deg > 0.0, lax.rsqrt(deg), 0.0)
    ssum = sp_ref[0] + sp_ref[1]
    h0v = h0_ref[...].astype(jnp.float32)
    t = (1.0 - ALPHA_C) * (dis * ssum) + ALPHA_C * h0v
    tw = jnp.dot(t, w_ref[...], preferred_element_type=jnp.float32,
                 precision=lax.Precision.HIGHEST)
    u = (1.0 - beta_l) * t + beta_l * tw
    u = g_ref[...] * u * (1.0 / math.sqrt(1.0 + EPS_C)) + bt_ref[...]
    h = jnp.maximum(u, 0.0)
    if last:
        out_ref[...] = jnp.dot(h, w1t_ref[...],
                               preferred_element_type=jnp.float32,
                               precision=lax.Precision.HIGHEST) + b1_ref[...]
    else:
        out_ref[...] = dis * h


def _row_spec(feat):
    return pl.BlockSpec((BK, feat), lambda i: (i, 0))


_FULL = pl.BlockSpec((FF, FF), lambda i: (0, 0))
_ROWV = pl.BlockSpec((1, FF), lambda i: (0, 0))
_DEGS = pl.BlockSpec((NC, BK, 1), lambda i: (0, i, 0))
_SPS = pl.BlockSpec((NC, BK, FF), lambda i: (0, i, 0))


def kernel(x, edge_index, W0, b0, W_convs, gammas, betas, W1, b1):
    row = edge_index[0].astype(jnp.int32)
    col = edge_index[1].astype(jnp.int32)
    pad = EP - EE
    pad_row = (jnp.arange(pad, dtype=jnp.int32) * 7) % NN
    pad_col = NN + (jnp.arange(pad, dtype=jnp.int32) % NPAD_ROWS)
    rowp = jnp.concatenate([row, pad_row])
    colp = jnp.concatenate([col, pad_col])
    rc3 = jnp.stack([rowp.reshape(NW * NBLK, BLK),
                     colp.reshape(NW * NBLK, BLK)], axis=1)
    colp3 = colp.reshape(NW, NBLK, BLK)

    degp = _deg_kernel(colp3)
    degp3 = degp.reshape(NC, NROWS, 1)

    h0, hs = pl.pallas_call(
        _first_body,
        grid=(GRID,),
        in_specs=[_row_spec(FF), _FULL, _ROWV, _DEGS],
        out_specs=[_row_spec(FF), _row_spec(FF)],
        out_shape=[jax.ShapeDtypeStruct((NN, FF), jnp.bfloat16),
                   jax.ShapeDtypeStruct((NN, FF), jnp.float32)],
    )(x, W0.T, b0.reshape(1, FF), degp3)

    dummy = jnp.zeros((FF, FF), jnp.float32)
    dummyv = jnp.zeros((1, FF), jnp.float32)
    w1t = W1.T
    b1v = b1.reshape(1, FF)

    for l in range(LL):
        sp = _spmm_kernel(hs, rc3)
        beta_l = math.log(THETA_C / (l + 1) + 1.0)
        last = l == LL - 1
        hs = pl.pallas_call(
            functools.partial(_layer_body, beta_l, last),
            grid=(GRID,),
            in_specs=[_SPS, _row_spec(FF), _DEGS, _FULL, _ROWV, _ROWV,
                      _FULL, _ROWV],
            out_specs=_row_spec(FF),
            out_shape=jax.ShapeDtypeStruct((NN, FF), jnp.float32),
        )(sp, h0, degp3, W_convs[l],
          gammas[l].reshape(1, FF), betas[l].reshape(1, FF),
          w1t if last else dummy, b1v if last else dummyv)
    return hs
